# baseline (speedup 1.0000x reference)
import jax
import jax.numpy as jnp
from jax.experimental import pallas as pl
from jax.experimental.pallas import tpu as pltpu

_NCHUNK = 16


def _kernel(x_ref, v_ref, m_ref, xo_ref, vo_ref, mo_ref, x_sems, v_sem):
    rows = x_ref.shape[0]
    ch = rows // _NCHUNK
    for i in range(_NCHUNK):
        pltpu.make_async_copy(
            x_ref.at[pl.ds(i * ch, ch)], xo_ref.at[pl.ds(i * ch, ch)], x_sems.at[i]
        ).start()
    pltpu.make_async_copy(v_ref, vo_ref, v_sem).start()
    mo_ref[...] = m_ref[...] != 0.0
    for i in range(_NCHUNK):
        pltpu.make_async_copy(
            x_ref.at[pl.ds(i * ch, ch)], xo_ref.at[pl.ds(i * ch, ch)], x_sems.at[i]
        ).wait()
    pltpu.make_async_copy(v_ref, vo_ref, v_sem).wait()


def kernel(x, v, mask):
    B, C, P = x.shape
    Vc = v.shape[1]
    x2 = x.reshape(B * C, P)
    v2 = v.reshape(B * Vc, P)
    m2 = mask.reshape(B, P)
    xo, vo, mo = pl.pallas_call(
        _kernel,
        in_specs=[
            pl.BlockSpec(memory_space=pl.ANY),
            pl.BlockSpec(memory_space=pl.ANY),
            pl.BlockSpec(memory_space=pltpu.VMEM),
        ],
        out_specs=[
            pl.BlockSpec(memory_space=pl.ANY),
            pl.BlockSpec(memory_space=pl.ANY),
            pl.BlockSpec(memory_space=pltpu.VMEM),
        ],
        out_shape=[
            jax.ShapeDtypeStruct((B * C, P), x.dtype),
            jax.ShapeDtypeStruct((B * Vc, P), v.dtype),
            jax.ShapeDtypeStruct((B, P), jnp.bool_),
        ],
        scratch_shapes=[
            pltpu.SemaphoreType.DMA((_NCHUNK,)),
            pltpu.SemaphoreType.DMA,
        ],
    )(x2, v2, m2)
    return (xo.reshape(B, C, P), vo.reshape(B, Vc, P), mo.reshape(B, 1, P))


# speedup vs baseline: 34.8669x; 34.8669x over previous
import jax
import jax.numpy as jnp
from jax.experimental import pallas as pl
from jax.experimental.pallas import tpu as pltpu

_NCH = 32
_K = 8


def _read(x_ref, buf, rsem, i, ch):
    s = i % _K
    return pltpu.make_async_copy(
        x_ref.at[pl.ds(i * ch, ch)], buf.at[s], rsem.at[s]
    )


def _write(buf, xo_ref, wsem, i, ch):
    s = i % _K
    return pltpu.make_async_copy(
        buf.at[s], xo_ref.at[pl.ds(i * ch, ch)], wsem.at[s]
    )


def _kernel(x_ref, v_ref, m_ref, xo_ref, vo_ref, mo_ref,
            buf, vbuf, rsem, wsem, vrsem, vwsem):
    rows = x_ref.shape[0]
    ch = rows // _NCH
    pltpu.make_async_copy(v_ref, vbuf, vrsem).start()
    for s in range(_K):
        _read(x_ref, buf, rsem, s, ch).start()
    mo_ref[...] = m_ref[...] != 0.0
    pltpu.make_async_copy(v_ref, vbuf, vrsem).wait()
    pltpu.make_async_copy(vbuf, vo_ref, vwsem).start()
    for i in range(_NCH):
        _read(x_ref, buf, rsem, i, ch).wait()
        _write(buf, xo_ref, wsem, i, ch).start()
        nxt = i + _K
        if nxt < _NCH:
            _write(buf, xo_ref, wsem, i, ch).wait()
            _read(x_ref, buf, rsem, nxt, ch).start()
    for i in range(_NCH - _K, _NCH):
        _write(buf, xo_ref, wsem, i, ch).wait()
    pltpu.make_async_copy(vbuf, vo_ref, vwsem).wait()


def kernel(x, v, mask):
    B, C, P = x.shape
    Vc = v.shape[1]
    rows = B * C
    ch = rows // _NCH
    x2 = x.reshape(rows, P)
    v2 = v.reshape(B * Vc, P)
    m2 = mask.reshape(B, P)
    xo, vo, mo = pl.pallas_call(
        _kernel,
        in_specs=[
            pl.BlockSpec(memory_space=pl.ANY),
            pl.BlockSpec(memory_space=pl.ANY),
            pl.BlockSpec(memory_space=pltpu.VMEM),
        ],
        out_specs=[
            pl.BlockSpec(memory_space=pl.ANY),
            pl.BlockSpec(memory_space=pl.ANY),
            pl.BlockSpec(memory_space=pltpu.VMEM),
        ],
        out_shape=[
            jax.ShapeDtypeStruct((rows, P), x.dtype),
            jax.ShapeDtypeStruct((B * Vc, P), v.dtype),
            jax.ShapeDtypeStruct((B, P), jnp.bool_),
        ],
        scratch_shapes=[
            pltpu.VMEM((_K, ch, P), x.dtype),
            pltpu.VMEM((B * Vc, P), v.dtype),
            pltpu.SemaphoreType.DMA((_K,)),
            pltpu.SemaphoreType.DMA((_K,)),
            pltpu.SemaphoreType.DMA,
            pltpu.SemaphoreType.DMA,
        ],
    )(x2, v2, m2)
    return (xo.reshape(B, C, P), vo.reshape(B, Vc, P), mo.reshape(B, 1, P))


# speedup vs baseline: 38.1083x; 1.0930x over previous
import jax
import jax.numpy as jnp
from jax.experimental import pallas as pl
from jax.experimental.pallas import tpu as pltpu

_NCH = 32
_S = 16
_D = 8


def _read(x_ref, buf, rsem, i, ch):
    s = i % _S
    return pltpu.make_async_copy(
        x_ref.at[pl.ds(i * ch, ch)], buf.at[s], rsem.at[s]
    )


def _write(buf, xo_ref, wsem, i, ch):
    s = i % _S
    return pltpu.make_async_copy(
        buf.at[s], xo_ref.at[pl.ds(i * ch, ch)], wsem.at[s]
    )


def _kernel(x_ref, v_ref, m_ref, xo_ref, vo_ref, mo_ref,
            buf, vbuf, rsem, wsem, vrsem, vwsem):
    rows = x_ref.shape[0]
    ch = rows // _NCH
    pltpu.make_async_copy(v_ref, vbuf, vrsem).start()
    for i in range(_D):
        _read(x_ref, buf, rsem, i, ch).start()
    mo_ref[...] = m_ref[...] != 0.0
    pltpu.make_async_copy(v_ref, vbuf, vrsem).wait()
    pltpu.make_async_copy(vbuf, vo_ref, vwsem).start()
    write_waited = [False] * _NCH
    for i in range(_NCH):
        _read(x_ref, buf, rsem, i, ch).wait()
        _write(buf, xo_ref, wsem, i, ch).start()
        j = i + _D
        if j < _NCH:
            prev = j - _S
            if prev >= 0:
                _write(buf, xo_ref, wsem, prev, ch).wait()
                write_waited[prev] = True
            _read(x_ref, buf, rsem, j, ch).start()
    for i in range(_NCH):
        if not write_waited[i]:
            _write(buf, xo_ref, wsem, i, ch).wait()
    pltpu.make_async_copy(vbuf, vo_ref, vwsem).wait()


def kernel(x, v, mask):
    B, C, P = x.shape
    Vc = v.shape[1]
    rows = B * C
    ch = rows // _NCH
    x2 = x.reshape(rows, P)
    v2 = v.reshape(B * Vc, P)
    m2 = mask.reshape(B, P)
    xo, vo, mo = pl.pallas_call(
        _kernel,
        in_specs=[
            pl.BlockSpec(memory_space=pl.ANY),
            pl.BlockSpec(memory_space=pl.ANY),
            pl.BlockSpec(memory_space=pltpu.VMEM),
        ],
        out_specs=[
            pl.BlockSpec(memory_space=pl.ANY),
            pl.BlockSpec(memory_space=pl.ANY),
            pl.BlockSpec(memory_space=pltpu.VMEM),
        ],
        out_shape=[
            jax.ShapeDtypeStruct((rows, P), x.dtype),
            jax.ShapeDtypeStruct((B * Vc, P), v.dtype),
            jax.ShapeDtypeStruct((B, P), jnp.bool_),
        ],
        scratch_shapes=[
            pltpu.VMEM((_S, ch, P), x.dtype),
            pltpu.VMEM((B * Vc, P), v.dtype),
            pltpu.SemaphoreType.DMA((_S,)),
            pltpu.SemaphoreType.DMA((_S,)),
            pltpu.SemaphoreType.DMA,
            pltpu.SemaphoreType.DMA,
        ],
    )(x2, v2, m2)
    return (xo.reshape(B, C, P), vo.reshape(B, Vc, P), mo.reshape(B, 1, P))


# speedup vs baseline: 38.6675x; 1.0147x over previous
import jax
import jax.numpy as jnp
from jax.experimental import pallas as pl
from jax.experimental.pallas import tpu as pltpu

_NCH = 8
_S = 4
_D = 2


def _read(x_ref, buf, rsem, i, ch):
    s = i % _S
    return pltpu.make_async_copy(
        x_ref.at[pl.ds(i * ch, ch)], buf.at[s], rsem.at[s]
    )


def _write(buf, xo_ref, wsem, i, ch):
    s = i % _S
    return pltpu.make_async_copy(
        buf.at[s], xo_ref.at[pl.ds(i * ch, ch)], wsem.at[s]
    )


def _kernel(x_ref, v_ref, m_ref, xo_ref, vo_ref, mo_ref,
            buf, vbuf, rsem, wsem, vrsem, vwsem):
    rows = x_ref.shape[0]
    ch = rows // _NCH
    pltpu.make_async_copy(v_ref, vbuf, vrsem).start()
    for i in range(_D):
        _read(x_ref, buf, rsem, i, ch).start()
    mo_ref[...] = m_ref[...] != 0.0
    pltpu.make_async_copy(v_ref, vbuf, vrsem).wait()
    pltpu.make_async_copy(vbuf, vo_ref, vwsem).start()
    write_waited = [False] * _NCH
    for i in range(_NCH):
        _read(x_ref, buf, rsem, i, ch).wait()
        _write(buf, xo_ref, wsem, i, ch).start()
        j = i + _D
        if j < _NCH:
            prev = j - _S
            if prev >= 0:
                _write(buf, xo_ref, wsem, prev, ch).wait()
                write_waited[prev] = True
            _read(x_ref, buf, rsem, j, ch).start()
    for i in range(_NCH):
        if not write_waited[i]:
            _write(buf, xo_ref, wsem, i, ch).wait()
    pltpu.make_async_copy(vbuf, vo_ref, vwsem).wait()


def kernel(x, v, mask):
    B, C, P = x.shape
    Vc = v.shape[1]
    rows = B * C
    ch = rows // _NCH
    x2 = x.reshape(rows, P)
    v2 = v.reshape(B * Vc, P)
    m2 = mask.reshape(B, P)
    xo, vo, mo = pl.pallas_call(
        _kernel,
        in_specs=[
            pl.BlockSpec(memory_space=pl.ANY),
            pl.BlockSpec(memory_space=pl.ANY),
            pl.BlockSpec(memory_space=pltpu.VMEM),
        ],
        out_specs=[
            pl.BlockSpec(memory_space=pl.ANY),
            pl.BlockSpec(memory_space=pl.ANY),
            pl.BlockSpec(memory_space=pltpu.VMEM),
        ],
        out_shape=[
            jax.ShapeDtypeStruct((rows, P), x.dtype),
            jax.ShapeDtypeStruct((B * Vc, P), v.dtype),
            jax.ShapeDtypeStruct((B, P), jnp.bool_),
        ],
        scratch_shapes=[
            pltpu.VMEM((_S, ch, P), x.dtype),
            pltpu.VMEM((B * Vc, P), v.dtype),
            pltpu.SemaphoreType.DMA((_S,)),
            pltpu.SemaphoreType.DMA((_S,)),
            pltpu.SemaphoreType.DMA,
            pltpu.SemaphoreType.DMA,
        ],
    )(x2, v2, m2)
    return (xo.reshape(B, C, P), vo.reshape(B, Vc, P), mo.reshape(B, 1, P))
